# speedup vs baseline: 1.0904x; 1.0904x over previous
import jax
import jax.numpy as jnp
from jax.experimental import pallas as pl


def _mm_body(ux_ref, ix_ref, uw_ref, iw_ref, ou_ref, oi_ref):
    ou_ref[...] = jnp.dot(ux_ref[...], uw_ref[...],
                          preferred_element_type=jnp.float32)
    oi_ref[...] = jnp.dot(ix_ref[...], iw_ref[...],
                          preferred_element_type=jnp.float32)


def kernel(user_x, item_x, ui_indices, ui_values, user_weight, item_weight):
    n_users, d_in = user_x.shape
    n_items = item_x.shape[0]
    d_out = user_weight.shape[1]
    blk = 1000
    xw_user, xw_item = pl.pallas_call(
        _mm_body,
        grid=(n_users // blk,),
        in_specs=[
            pl.BlockSpec((blk, d_in), lambda i: (i, 0)),
            pl.BlockSpec((blk, d_in), lambda i: (i, 0)),
            pl.BlockSpec((d_in, d_out), lambda i: (0, 0)),
            pl.BlockSpec((d_in, d_out), lambda i: (0, 0)),
        ],
        out_specs=[
            pl.BlockSpec((blk, d_out), lambda i: (i, 0)),
            pl.BlockSpec((blk, d_out), lambda i: (i, 0)),
        ],
        out_shape=[
            jax.ShapeDtypeStruct((n_users, d_out), jnp.float32),
            jax.ShapeDtypeStruct((n_items, d_out), jnp.float32),
        ],
    )(user_x, item_x, user_weight, item_weight)

    rows = ui_indices[0]
    cols = ui_indices[1]
    msgs_u = jnp.take(xw_item, cols, axis=0) * ui_values[:, None]
    out_user = jax.ops.segment_sum(msgs_u, rows, num_segments=n_users)
    msgs_i = jnp.take(xw_user, rows, axis=0) * ui_values[:, None]
    out_item = jax.ops.segment_sum(msgs_i, cols, num_segments=n_items)
    return (jax.nn.relu(out_user), jax.nn.relu(out_item))


# speedup vs baseline: 3.0546x; 2.8014x over previous
import jax
import jax.numpy as jnp
from jax import lax
from jax.experimental import pallas as pl
from jax.experimental.pallas import tpu as pltpu
from jax.experimental.pallas import tpu_sc as plsc

N_NODES = 10000
N_PAD = 10240
D = 128
E = 320000
NSC = 16
LANES = 16
CHUNK = 80
EPS = E // NSC
NCHUNKS = EPS // CHUNK
ROW_BLK = 128
RPS = N_PAD // NSC


def _mm_body(ux_ref, ix_ref, uw_ref, iw_ref, ou_ref, oi_ref):
    ou_ref[...] = jnp.dot(ux_ref[...], uw_ref[...],
                          preferred_element_type=jnp.float32)
    oi_ref[...] = jnp.dot(ix_ref[...], iw_ref[...],
                          preferred_element_type=jnp.float32)


def _project(user_x, item_x, user_weight, item_weight):
    n, d_in = user_x.shape
    d_out = user_weight.shape[1]
    blk = 1000
    return pl.pallas_call(
        _mm_body,
        grid=(n // blk,),
        in_specs=[
            pl.BlockSpec((blk, d_in), lambda i: (i, 0)),
            pl.BlockSpec((blk, d_in), lambda i: (i, 0)),
            pl.BlockSpec((d_in, d_out), lambda i: (0, 0)),
            pl.BlockSpec((d_in, d_out), lambda i: (0, 0)),
        ],
        out_specs=[
            pl.BlockSpec((blk, d_out), lambda i: (i, 0)),
            pl.BlockSpec((blk, d_out), lambda i: (i, 0)),
        ],
        out_shape=[
            jax.ShapeDtypeStruct((n, d_out), jnp.float32),
            jax.ShapeDtypeStruct((n, d_out), jnp.float32),
        ],
    )(user_x, item_x, user_weight, item_weight)


def _sc_body(xwu_hbm, xwi_hbm, rows_hbm, cols_hbm, vals_hbm,
             outu_hbm, outi_hbm,
             isrc_v, idst_v, valb_v, msg_v, stage_v, acc_sh, sem):
    cid = lax.axis_index("c")
    sid = lax.axis_index("s")

    @pl.loop(0, ROW_BLK)
    def _(r):
        for b in range(D // LANES):
            stage_v.at[pl.ds(r, 1), pl.ds(b * LANES, LANES)][...] = (
                jnp.zeros((1, LANES), jnp.float32))
    for k in range(RPS // ROW_BLK):
        pltpu.sync_copy(stage_v, acc_sh.at[pl.ds(sid * RPS + k * ROW_BLK,
                                                 ROW_BLK)])
    plsc.subcore_barrier()

    def accumulate(table_hbm, src_hbm, dst_hbm):
        @pl.loop(0, NCHUNKS)
        def _(c):
            base = sid * EPS + c * CHUNK
            pltpu.sync_copy(src_hbm.at[pl.ds(base, CHUNK)], isrc_v)
            pltpu.sync_copy(dst_hbm.at[pl.ds(base, CHUNK)], idst_v)
            pltpu.sync_copy(vals_hbm.at[pl.ds(base, CHUNK)], valb_v)
            pltpu.async_copy(table_hbm.at[isrc_v], msg_v, sem).wait()

            @pl.loop(0, CHUNK)
            def _(j):
                srow = valb_v.at[pl.ds(j, 1), pl.ds(0, LANES)][...]
                for b in range(D // LANES):
                    slc = (pl.ds(j, 1), pl.ds(b * LANES, LANES))
                    msg_v.at[slc][...] = msg_v.at[slc][...] * srow

            pltpu.sync_copy(msg_v, acc_sh.at[idst_v], add=True)

    @pl.when(cid == 0)
    def _():
        accumulate(xwi_hbm, cols_hbm, rows_hbm)

    @pl.when(cid == 1)
    def _():
        accumulate(xwu_hbm, rows_hbm, cols_hbm)

    plsc.subcore_barrier()

    def writeback(out_hbm):
        for k in range(RPS // ROW_BLK):
            r0 = sid * RPS + k * ROW_BLK
            pltpu.sync_copy(acc_sh.at[pl.ds(r0, ROW_BLK)], stage_v)

            @pl.loop(0, ROW_BLK)
            def _(r):
                for b in range(D // LANES):
                    slc = (pl.ds(r, 1), pl.ds(b * LANES, LANES))
                    stage_v.at[slc][...] = jnp.maximum(stage_v.at[slc][...],
                                                       0.0)
            pltpu.sync_copy(stage_v, out_hbm.at[pl.ds(r0, ROW_BLK)])

    @pl.when(cid == 0)
    def _():
        writeback(outu_hbm)

    @pl.when(cid == 1)
    def _():
        writeback(outi_hbm)


def kernel(user_x, item_x, ui_indices, ui_values, user_weight, item_weight):
    xw_user, xw_item = _project(user_x, item_x, user_weight, item_weight)
    rows = ui_indices[0].astype(jnp.int32)
    cols = ui_indices[1].astype(jnp.int32)

    mesh = plsc.VectorSubcoreMesh(core_axis_name="c", subcore_axis_name="s")
    sc_fn = pl.kernel(
        _sc_body,
        out_type=[
            jax.ShapeDtypeStruct((N_PAD, D), jnp.float32),
            jax.ShapeDtypeStruct((N_PAD, D), jnp.float32),
        ],
        mesh=mesh,
        scratch_types=[
            pltpu.VMEM((CHUNK,), jnp.int32),
            pltpu.VMEM((CHUNK,), jnp.int32),
            pltpu.VMEM((CHUNK, LANES), jnp.float32),
            pltpu.VMEM((CHUNK, D), jnp.float32),
            pltpu.VMEM((ROW_BLK, D), jnp.float32),
            pltpu.VMEM_SHARED((N_PAD, D), jnp.float32),
            pltpu.SemaphoreType.DMA,
        ],
    )
    vals_bcast = jnp.broadcast_to(ui_values[:, None], (E, LANES))
    out_user, out_item = sc_fn(xw_user, xw_item, rows, cols, vals_bcast)
    return (out_user[:N_NODES], out_item[:N_NODES])


# speedup vs baseline: 7.1654x; 2.3457x over previous
import jax
import jax.numpy as jnp
from jax import lax
from jax.experimental import pallas as pl
from jax.experimental.pallas import tpu as pltpu
from jax.experimental.pallas import tpu_sc as plsc

N_NODES = 10000
N_PAD = 10240
D = 128
E = 320000
NSC = 16
LANES = 16
CHUNK = 80
VBATCH = 10
NVB = 25
EPS = NVB * VBATCH * CHUNK
BE = VBATCH * CHUNK
ROW_BLK = 80
RPS = N_PAD // NSC


def _mm_body(ux_ref, ix_ref, uw_ref, iw_ref, ou_ref, oi_ref):
    ou_ref[...] = jnp.dot(ux_ref[...], uw_ref[...],
                          preferred_element_type=jnp.float32)
    oi_ref[...] = jnp.dot(ix_ref[...], iw_ref[...],
                          preferred_element_type=jnp.float32)


def _project(user_x, item_x, user_weight, item_weight):
    n, d_in = user_x.shape
    d_out = user_weight.shape[1]
    blk = 1000
    return pl.pallas_call(
        _mm_body,
        grid=(n // blk,),
        in_specs=[
            pl.BlockSpec((blk, d_in), lambda i: (i, 0)),
            pl.BlockSpec((blk, d_in), lambda i: (i, 0)),
            pl.BlockSpec((d_in, d_out), lambda i: (0, 0)),
            pl.BlockSpec((d_in, d_out), lambda i: (0, 0)),
        ],
        out_specs=[
            pl.BlockSpec((blk, d_out), lambda i: (i, 0)),
            pl.BlockSpec((blk, d_out), lambda i: (i, 0)),
        ],
        out_shape=[
            jax.ShapeDtypeStruct((n, d_out), jnp.float32),
            jax.ShapeDtypeStruct((n, d_out), jnp.float32),
        ],
    )(user_x, item_x, user_weight, item_weight)


def _sc_body(xwu_hbm, xwi_hbm, rows_hbm, cols_hbm, vals_hbm,
             outu_hbm, outi_hbm,
             isrc_v, idst_v, valb_v, msga_v, msgb_v, acc_sh,
             sema, semb):
    cid = lax.axis_index("c")
    sid = lax.axis_index("s")

    @pl.loop(0, ROW_BLK)
    def _(r):
        for b in range(D // LANES):
            msga_v.at[pl.ds(r, 1), pl.ds(b * LANES, LANES)][...] = (
                jnp.zeros((1, LANES), jnp.float32))
    for k in range(RPS // ROW_BLK):
        pltpu.sync_copy(msga_v, acc_sh.at[pl.ds(sid * RPS + k * ROW_BLK,
                                                ROW_BLK)])
    plsc.subcore_barrier()

    def scale(msg_v, k):
        @pl.loop(0, CHUNK, step=8)
        def _(j0):
            row = 10 * k + j0 // 8
            for u in range(8):
                srow = valb_v.at[pl.ds(row, 1),
                                 pl.ds(u * LANES, LANES)][...]
                for b in range(D // LANES):
                    slc = (pl.ds(j0 + u, 1), pl.ds(b * LANES, LANES))
                    msg_v.at[slc][...] = msg_v.at[slc][...] * srow

    def accumulate(table_hbm, src_hbm, dst_hbm):
        @pl.loop(0, NVB)
        def _(t):
            pltpu.sync_copy(src_hbm.at[sid, t], isrc_v)
            ha = pltpu.async_copy(table_hbm.at[isrc_v.at[0]], msga_v, sema)
            pltpu.sync_copy(dst_hbm.at[sid, t], idst_v)
            pltpu.sync_copy(vals_hbm.at[sid, t], valb_v)
            for k in range(VBATCH):
                cur = msga_v if k % 2 == 0 else msgb_v
                nxt = msgb_v if k % 2 == 0 else msga_v
                scur = sema if k % 2 == 0 else semb
                snxt = semb if k % 2 == 0 else sema
                if k + 1 < VBATCH:
                    pltpu.async_copy(table_hbm.at[isrc_v.at[k + 1]], nxt,
                                     snxt)
                pltpu.make_async_copy(table_hbm.at[isrc_v.at[k]], cur,
                                      scur).wait()
                scale(cur, k)
                pltpu.sync_copy(cur, acc_sh.at[idst_v.at[k]], add=True)

    @pl.when(cid == 0)
    def _():
        accumulate(xwi_hbm, cols_hbm, rows_hbm)

    @pl.when(cid == 1)
    def _():
        accumulate(xwu_hbm, rows_hbm, cols_hbm)

    plsc.subcore_barrier()

    def writeback(out_hbm):
        for k in range(RPS // ROW_BLK):
            r0 = sid * RPS + k * ROW_BLK
            pltpu.sync_copy(acc_sh.at[pl.ds(r0, ROW_BLK)], msga_v)

            @pl.loop(0, ROW_BLK)
            def _(r):
                for b in range(D // LANES):
                    slc = (pl.ds(r, 1), pl.ds(b * LANES, LANES))
                    msga_v.at[slc][...] = jnp.maximum(msga_v.at[slc][...],
                                                      0.0)
            pltpu.sync_copy(msga_v, out_hbm.at[pl.ds(r0, ROW_BLK)])

    @pl.when(cid == 0)
    def _():
        writeback(outu_hbm)

    @pl.when(cid == 1)
    def _():
        writeback(outi_hbm)


def kernel(user_x, item_x, ui_indices, ui_values, user_weight, item_weight):
    xw_user, xw_item = _project(user_x, item_x, user_weight, item_weight)
    rows = ui_indices[0].astype(jnp.int32)
    cols = ui_indices[1].astype(jnp.int32)
    rows4 = rows.reshape(NSC, NVB, VBATCH, CHUNK)
    cols4 = cols.reshape(NSC, NVB, VBATCH, CHUNK)
    vals4 = jnp.broadcast_to(ui_values[:, None], (E, LANES)).reshape(
        NSC, NVB, BE // 8, 8 * LANES)

    mesh = plsc.VectorSubcoreMesh(core_axis_name="c", subcore_axis_name="s")
    sc_fn = pl.kernel(
        _sc_body,
        out_type=[
            jax.ShapeDtypeStruct((N_PAD, D), jnp.float32),
            jax.ShapeDtypeStruct((N_PAD, D), jnp.float32),
        ],
        mesh=mesh,
        scratch_types=[
            pltpu.VMEM((VBATCH, CHUNK), jnp.int32),
            pltpu.VMEM((VBATCH, CHUNK), jnp.int32),
            pltpu.VMEM((BE // 8, 8 * LANES), jnp.float32),
            pltpu.VMEM((CHUNK, D), jnp.float32),
            pltpu.VMEM((CHUNK, D), jnp.float32),
            pltpu.VMEM_SHARED((N_PAD, D), jnp.float32),
            pltpu.SemaphoreType.DMA,
            pltpu.SemaphoreType.DMA,
        ],
    )
    out_user, out_item = sc_fn(xw_user, xw_item, rows4, cols4, vals4)
    return (out_user[:N_NODES], out_item[:N_NODES])
